# baseline (speedup 1.0000x reference)
import functools

import jax
import jax.numpy as jnp
from jax import lax
from jax.experimental import pallas as pl
from jax.experimental.pallas import tpu as pltpu
from jax.experimental.pallas import tpu_sc as plsc

H = 64
W = 72
CHUNK = 128
NUM_LAYERS = 2
N = {'network': 500, 'host': 25000, 'service': 12500, 'data': 12000}
RELS = [('hn', 'host', 'network', 50000), ('nh', 'network', 'host', 50000),
        ('hs', 'host', 'service', 200000), ('hd', 'host', 'data', 200000),
        ('sh', 'service', 'host', 150000), ('dh', 'data', 'host', 150000)]

_N_TILES = 32


def _ceil_to(x, m):
    return (x + m - 1) // m * m


@functools.lru_cache(maxsize=None)
def _edge_kernel(e_pad, n_acc, chunk):
    per_tile = e_pad // _N_TILES
    n_chunks = per_tile // chunk
    rows_pt = n_acc // 16
    mesh = plsc.VectorSubcoreMesh(core_axis_name="c", subcore_axis_name="s")

    def body(hl_hbm, hr_hbm, src_hbm, dst_hbm, att_hbm, out_hbm,
             acc_sh, src_a, dst_a, hl_a, hr_a, src_b, dst_b, hl_b, hr_b,
             att_v, zero_v, sa0, sa1, sb0, sb1):
        c = lax.axis_index("c")
        s = lax.axis_index("s")
        pltpu.sync_copy(att_hbm, att_v)
        zeros16 = jnp.zeros((16,), jnp.float32)

        def zrow(i, carry):
            for off in (0, 16, 32, 48, W - 16):
                zero_v[i, pl.ds(off, 16)] = zeros16
            return carry
        lax.fori_loop(0, 16, zrow, 0)

        row0 = s * rows_pt

        def zacc(j, carry):
            pltpu.sync_copy(zero_v, acc_sh.at[pl.ds(row0 + j * 16, 16)])
            return carry
        lax.fori_loop(0, rows_pt // 16, zacc, 0)
        plsc.subcore_barrier()

        tile_base = (c * 16 + s) * per_tile
        eidx0 = lax.iota(jnp.int32, 16)
        att_vecs = [att_v[pl.ds(k * 16, 16)] for k in range(H // 16)]

        sides = ((src_a, dst_a, hl_a, hr_a, sa0, sa1),
                 (src_b, dst_b, hl_b, hr_b, sb0, sb1))

        def issue(side, ci):
            src_v, dst_v, hl_rows, hr_rows, s0, s1 = side
            base = tile_base + ci * chunk
            pltpu.sync_copy(src_hbm.at[pl.ds(base, chunk)], src_v)
            pltpu.sync_copy(dst_hbm.at[pl.ds(base, chunk)], dst_v)
            pltpu.async_copy(hl_hbm.at[src_v], hl_rows, s0)
            pltpu.async_copy(hr_hbm.at[dst_v], hr_rows, s1)

        def wait_side(side):
            src_v, dst_v, hl_rows, hr_rows, s0, s1 = side
            pltpu.make_async_copy(hl_hbm.at[src_v], hl_rows, s0).wait()
            pltpu.make_async_copy(hr_hbm.at[dst_v], hr_rows, s1).wait()

        def compute_scatter(side):
            src_v, dst_v, hl_rows, hr_rows, s0, s1 = side

            def group(g, carry2):
                e_idx = eidx0 + g * 16
                exs = jnp.zeros((16,), jnp.float32)
                for e in range(16):
                    e_abs = g * 16 + e
                    avs = [hl_rows[e_abs, pl.ds(k * 16, 16)]
                           for k in range(H // 16)]
                    ps = []
                    for k in range(H // 16):
                        sm = avs[k] + hr_rows[e_abs, pl.ds(k * 16, 16)]
                        z = jnp.maximum(sm, 0.2 * sm)
                        ps.append(z * att_vecs[k])
                    t = (ps[0] + ps[1]) + (ps[2] + ps[3])
                    logit = jnp.sum(t)
                    ex = jnp.exp(jnp.broadcast_to(logit, (16,)))
                    for k in range(H // 16):
                        hl_rows[e_abs, pl.ds(k * 16, 16)] = avs[k] * ex
                    exs = jnp.where(eidx0 == e, ex, exs)
                f_idx = jnp.full((16,), H, jnp.int32)
                plsc.store_scatter(hl_rows, [e_idx, f_idx], exs)
                return carry2
            lax.fori_loop(0, chunk // 16, group, 0)
            pltpu.sync_copy(hl_rows, acc_sh.at[dst_v], add=True)

        n_half = n_chunks // 2
        issue(sides[0], 0)

        def pair_body(i2, carry):
            wait_side(sides[0])
            issue(sides[1], 2 * i2 + 1)
            compute_scatter(sides[0])
            wait_side(sides[1])

            @pl.when(i2 < n_half - 1)
            def _():
                issue(sides[0], 2 * i2 + 2)
            compute_scatter(sides[1])
            return carry
        lax.fori_loop(0, n_half, pair_body, 0)
        plsc.subcore_barrier()

        def drain(j, carry):
            r = row0 + j * 16
            pltpu.sync_copy(acc_sh.at[pl.ds(r, 16)], out_hbm.at[c, pl.ds(r, 16)])
            return carry
        lax.fori_loop(0, rows_pt // 16, drain, 0)

    return pl.kernel(
        body,
        out_type=jax.ShapeDtypeStruct((2, n_acc, W), jnp.float32),
        mesh=mesh,
        compiler_params=pltpu.CompilerParams(
            needs_layout_passes=False, use_tc_tiling_on_sc=False),
        scratch_types=[
            pltpu.VMEM_SHARED((n_acc, W), jnp.float32),
            pltpu.VMEM((chunk,), jnp.int32),
            pltpu.VMEM((chunk,), jnp.int32),
            pltpu.VMEM((chunk, W), jnp.float32),
            pltpu.VMEM((chunk, H), jnp.float32),
            pltpu.VMEM((chunk,), jnp.int32),
            pltpu.VMEM((chunk,), jnp.int32),
            pltpu.VMEM((chunk, W), jnp.float32),
            pltpu.VMEM((chunk, H), jnp.float32),
            pltpu.VMEM((H,), jnp.float32),
            pltpu.VMEM((16, W), jnp.float32),
            pltpu.SemaphoreType.DMA,
            pltpu.SemaphoreType.DMA,
            pltpu.SemaphoreType.DMA,
            pltpu.SemaphoreType.DMA,
        ],
    )


def _gatv2_sc(h_src, h_dst, src, dst, Wl, Wr, att, b, num_dst):
    n_src = h_src.shape[0]
    e = src.shape[0]
    n_acc = _ceil_to(num_dst + 1, 256)
    chunk = 48 if n_acc * W > 1500 * 1024 else CHUNK
    e_pad = _ceil_to(e, 2 * _N_TILES * chunk)

    hl = h_src @ Wl
    hl_ext = jnp.concatenate(
        [hl, jnp.ones((n_src, 1), jnp.float32),
         jnp.zeros((n_src, W - H - 1), jnp.float32)], axis=1)
    hr = h_dst @ Wr
    src_p = jnp.concatenate([src, jnp.zeros((e_pad - e,), jnp.int32)])
    dst_p = jnp.concatenate(
        [dst, jnp.full((e_pad - e,), num_dst, jnp.int32)])

    acc = _edge_kernel(e_pad, n_acc, chunk)(hl_ext, hr, src_p, dst_p, att)
    tot = acc[0] + acc[1]
    den = tot[:num_dst, H]
    out = tot[:num_dst, :H] / (den + 1e-16)[:, None]
    return out + b


def kernel(x_network, x_host, x_service, x_data, e_hn_src, e_hn_dst, e_nh_src, e_nh_dst, e_hs_src, e_hs_dst, e_hd_src, e_hd_dst, e_sh_src, e_sh_dst, e_dh_src, e_dh_dst, enc_W_network, enc_b_network, enc_W_host, enc_b_host, enc_W_service, enc_b_service, enc_W_data, enc_b_data, conv0_hn_Wl, conv0_hn_Wr, conv0_hn_att, conv0_hn_b, conv0_nh_Wl, conv0_nh_Wr, conv0_nh_att, conv0_nh_b, conv0_hs_Wl, conv0_hs_Wr, conv0_hs_att, conv0_hs_b, conv0_hd_Wl, conv0_hd_Wr, conv0_hd_att, conv0_hd_b, conv0_sh_Wl, conv0_sh_Wr, conv0_sh_att, conv0_sh_b, conv0_dh_Wl, conv0_dh_Wr, conv0_dh_att, conv0_dh_b, conv1_hn_Wl, conv1_hn_Wr, conv1_hn_att, conv1_hn_b, conv1_nh_Wl, conv1_nh_Wr, conv1_nh_att, conv1_nh_b, conv1_hs_Wl, conv1_hs_Wr, conv1_hs_att, conv1_hs_b, conv1_hd_Wl, conv1_hd_Wr, conv1_hd_att, conv1_hd_b, conv1_sh_Wl, conv1_sh_Wr, conv1_sh_att, conv1_sh_b, conv1_dh_Wl, conv1_dh_Wr, conv1_dh_att, conv1_dh_b, val_W1, val_b1, val_W2, val_b2, state_summary):
    fl = dict(locals())
    it = {}
    for rel, s, d, e in RELS:
        it['e_' + rel + '_src'] = fl.pop('e_' + rel + '_src')
        it['e_' + rel + '_dst'] = fl.pop('e_' + rel + '_dst')

    h = {}
    for nt in N:
        h[nt] = jax.nn.relu(fl['x_' + nt] @ fl['enc_W_' + nt] + fl['enc_b_' + nt])
    for l in range(NUM_LAYERS):
        new = {nt: jnp.zeros_like(h[nt]) for nt in N}
        for rel, s, d, e in RELS:
            p = 'conv%d_%s_' % (l, rel)
            out = _gatv2_sc(h[s], h[d], it['e_' + rel + '_src'],
                            it['e_' + rel + '_dst'],
                            fl[p + 'Wl'], fl[p + 'Wr'], fl[p + 'att'],
                            fl[p + 'b'], N[d])
            new[d] = new[d] + out
        h = {nt: jax.nn.relu(new[nt]) for nt in N}
    g = jnp.stack([h[nt].mean(axis=0) for nt in N]).mean(axis=0)
    z = jnp.concatenate([g, state_summary])
    v = jax.nn.relu(z @ val_W1 + val_b1) @ val_W2 + val_b2
    return v.squeeze(-1)


# speedup vs baseline: 1.0140x; 1.0140x over previous
import functools

import jax
import jax.numpy as jnp
from jax import lax
from jax.experimental import pallas as pl
from jax.experimental.pallas import tpu as pltpu
from jax.experimental.pallas import tpu_sc as plsc

H = 64
W = 72
CHUNK = 128
NUM_LAYERS = 2
N = {'network': 500, 'host': 25000, 'service': 12500, 'data': 12000}
RELS = [('hn', 'host', 'network', 50000), ('nh', 'network', 'host', 50000),
        ('hs', 'host', 'service', 200000), ('hd', 'host', 'data', 200000),
        ('sh', 'service', 'host', 150000), ('dh', 'data', 'host', 150000)]

_N_TILES = 32


def _ceil_to(x, m):
    return (x + m - 1) // m * m


@functools.lru_cache(maxsize=None)
def _edge_kernel(e_pad, n_acc, chunk, nbuf):
    per_tile = e_pad // _N_TILES
    n_chunks = per_tile // chunk
    rows_pt = n_acc // 16
    mesh = plsc.VectorSubcoreMesh(core_axis_name="c", subcore_axis_name="s")

    def body(hl_hbm, hr_hbm, src_hbm, dst_hbm, att_hbm, out_hbm,
             acc_sh, *scr):
        buf_refs, rest = scr[:4 * nbuf], scr[4 * nbuf:]
        att_v, zero_v = rest[0], rest[1]
        sems = rest[2:]
        c = lax.axis_index("c")
        s = lax.axis_index("s")
        pltpu.sync_copy(att_hbm, att_v)
        zeros16 = jnp.zeros((16,), jnp.float32)

        def zrow(i, carry):
            for off in (0, 16, 32, 48, W - 16):
                zero_v[i, pl.ds(off, 16)] = zeros16
            return carry
        lax.fori_loop(0, 16, zrow, 0)

        row0 = s * rows_pt

        def zacc(j, carry):
            pltpu.sync_copy(zero_v, acc_sh.at[pl.ds(row0 + j * 16, 16)])
            return carry
        lax.fori_loop(0, rows_pt // 16, zacc, 0)
        plsc.subcore_barrier()

        tile_base = (c * 16 + s) * per_tile
        eidx0 = lax.iota(jnp.int32, 16)
        att_vecs = [att_v[pl.ds(k * 16, 16)] for k in range(H // 16)]

        sides = tuple(
            buf_refs[4 * j:4 * j + 4] + sems[2 * j:2 * j + 2]
            for j in range(nbuf))

        def issue(side, ci):
            src_v, dst_v, hl_rows, hr_rows, s0, s1 = side
            base = tile_base + ci * chunk
            pltpu.sync_copy(src_hbm.at[pl.ds(base, chunk)], src_v)
            pltpu.sync_copy(dst_hbm.at[pl.ds(base, chunk)], dst_v)
            pltpu.async_copy(hl_hbm.at[src_v], hl_rows, s0)
            pltpu.async_copy(hr_hbm.at[dst_v], hr_rows, s1)

        def wait_side(side):
            src_v, dst_v, hl_rows, hr_rows, s0, s1 = side
            pltpu.make_async_copy(hl_hbm.at[src_v], hl_rows, s0).wait()
            pltpu.make_async_copy(hr_hbm.at[dst_v], hr_rows, s1).wait()

        def compute_scatter(side):
            src_v, dst_v, hl_rows, hr_rows, s0, s1 = side

            def group(g, carry2):
                e_idx = eidx0 + g * 16
                exs = jnp.zeros((16,), jnp.float32)
                for e in range(16):
                    e_abs = g * 16 + e
                    avs = [hl_rows[e_abs, pl.ds(k * 16, 16)]
                           for k in range(H // 16)]
                    ps = []
                    for k in range(H // 16):
                        sm = avs[k] + hr_rows[e_abs, pl.ds(k * 16, 16)]
                        z = jnp.maximum(sm, 0.2 * sm)
                        ps.append(z * att_vecs[k])
                    t = (ps[0] + ps[1]) + (ps[2] + ps[3])
                    logit = jnp.sum(t)
                    ex = jnp.exp(jnp.broadcast_to(logit, (16,)))
                    for k in range(H // 16):
                        hl_rows[e_abs, pl.ds(k * 16, 16)] = avs[k] * ex
                    exs = jnp.where(eidx0 == e, ex, exs)
                f_idx = jnp.full((16,), H, jnp.int32)
                plsc.store_scatter(hl_rows, [e_idx, f_idx], exs)
                return carry2
            lax.fori_loop(0, chunk // 16, group, 0)
            pltpu.sync_copy(hl_rows, acc_sh.at[dst_v], add=True)

        if nbuf == 2:
            n_half = n_chunks // 2
            issue(sides[0], 0)

            def pair_body(i2, carry):
                wait_side(sides[0])
                issue(sides[1], 2 * i2 + 1)
                compute_scatter(sides[0])
                wait_side(sides[1])

                @pl.when(i2 < n_half - 1)
                def _():
                    issue(sides[0], 2 * i2 + 2)
                compute_scatter(sides[1])
                return carry
            lax.fori_loop(0, n_half, pair_body, 0)
        else:
            def chunk_body(i, carry):
                issue(sides[0], i)
                wait_side(sides[0])
                compute_scatter(sides[0])
                return carry
            lax.fori_loop(0, n_chunks, chunk_body, 0)
        plsc.subcore_barrier()

        def drain(j, carry):
            r = row0 + j * 16
            pltpu.sync_copy(acc_sh.at[pl.ds(r, 16)], out_hbm.at[c, pl.ds(r, 16)])
            return carry
        lax.fori_loop(0, rows_pt // 16, drain, 0)

    return pl.kernel(
        body,
        out_type=jax.ShapeDtypeStruct((2, n_acc, W), jnp.float32),
        mesh=mesh,
        compiler_params=pltpu.CompilerParams(
            needs_layout_passes=False, use_tc_tiling_on_sc=False),
        scratch_types=(
            [pltpu.VMEM_SHARED((n_acc, W), jnp.float32)]
            + [pltpu.VMEM((chunk,), jnp.int32),
               pltpu.VMEM((chunk,), jnp.int32),
               pltpu.VMEM((chunk, W), jnp.float32),
               pltpu.VMEM((chunk, H), jnp.float32)] * nbuf
            + [pltpu.VMEM((H,), jnp.float32),
               pltpu.VMEM((16, W), jnp.float32)]
            + [pltpu.SemaphoreType.DMA] * (2 * nbuf)
        ),
    )


def _gatv2_sc(h_src, h_dst, src, dst, Wl, Wr, att, b, num_dst):
    n_src = h_src.shape[0]
    e = src.shape[0]
    n_acc = _ceil_to(num_dst + 1, 256)
    big_acc = n_acc * W > 1500 * 1024
    chunk = 112 if big_acc else CHUNK
    nbuf = 1 if big_acc else 2
    e_pad = _ceil_to(e, nbuf * _N_TILES * chunk)

    hl = h_src @ Wl
    hl_ext = jnp.concatenate(
        [hl, jnp.ones((n_src, 1), jnp.float32),
         jnp.zeros((n_src, W - H - 1), jnp.float32)], axis=1)
    hr = h_dst @ Wr
    src_p = jnp.concatenate([src, jnp.zeros((e_pad - e,), jnp.int32)])
    dst_p = jnp.concatenate(
        [dst, jnp.full((e_pad - e,), num_dst, jnp.int32)])

    acc = _edge_kernel(e_pad, n_acc, chunk, nbuf)(hl_ext, hr, src_p, dst_p, att)
    tot = acc[0] + acc[1]
    den = tot[:num_dst, H]
    out = tot[:num_dst, :H] / (den + 1e-16)[:, None]
    return out + b


def kernel(x_network, x_host, x_service, x_data, e_hn_src, e_hn_dst, e_nh_src, e_nh_dst, e_hs_src, e_hs_dst, e_hd_src, e_hd_dst, e_sh_src, e_sh_dst, e_dh_src, e_dh_dst, enc_W_network, enc_b_network, enc_W_host, enc_b_host, enc_W_service, enc_b_service, enc_W_data, enc_b_data, conv0_hn_Wl, conv0_hn_Wr, conv0_hn_att, conv0_hn_b, conv0_nh_Wl, conv0_nh_Wr, conv0_nh_att, conv0_nh_b, conv0_hs_Wl, conv0_hs_Wr, conv0_hs_att, conv0_hs_b, conv0_hd_Wl, conv0_hd_Wr, conv0_hd_att, conv0_hd_b, conv0_sh_Wl, conv0_sh_Wr, conv0_sh_att, conv0_sh_b, conv0_dh_Wl, conv0_dh_Wr, conv0_dh_att, conv0_dh_b, conv1_hn_Wl, conv1_hn_Wr, conv1_hn_att, conv1_hn_b, conv1_nh_Wl, conv1_nh_Wr, conv1_nh_att, conv1_nh_b, conv1_hs_Wl, conv1_hs_Wr, conv1_hs_att, conv1_hs_b, conv1_hd_Wl, conv1_hd_Wr, conv1_hd_att, conv1_hd_b, conv1_sh_Wl, conv1_sh_Wr, conv1_sh_att, conv1_sh_b, conv1_dh_Wl, conv1_dh_Wr, conv1_dh_att, conv1_dh_b, val_W1, val_b1, val_W2, val_b2, state_summary):
    fl = dict(locals())
    it = {}
    for rel, s, d, e in RELS:
        it['e_' + rel + '_src'] = fl.pop('e_' + rel + '_src')
        it['e_' + rel + '_dst'] = fl.pop('e_' + rel + '_dst')

    h = {}
    for nt in N:
        h[nt] = jax.nn.relu(fl['x_' + nt] @ fl['enc_W_' + nt] + fl['enc_b_' + nt])
    for l in range(NUM_LAYERS):
        new = {nt: jnp.zeros_like(h[nt]) for nt in N}
        for rel, s, d, e in RELS:
            p = 'conv%d_%s_' % (l, rel)
            out = _gatv2_sc(h[s], h[d], it['e_' + rel + '_src'],
                            it['e_' + rel + '_dst'],
                            fl[p + 'Wl'], fl[p + 'Wr'], fl[p + 'att'],
                            fl[p + 'b'], N[d])
            new[d] = new[d] + out
        h = {nt: jax.nn.relu(new[nt]) for nt in N}
    g = jnp.stack([h[nt].mean(axis=0) for nt in N]).mean(axis=0)
    z = jnp.concatenate([g, state_summary])
    v = jax.nn.relu(z @ val_W1 + val_b1) @ val_W2 + val_b2
    return v.squeeze(-1)


# speedup vs baseline: 1.0230x; 1.0089x over previous
import functools

import jax
import jax.numpy as jnp
from jax import lax
from jax.experimental import pallas as pl
from jax.experimental.pallas import tpu as pltpu
from jax.experimental.pallas import tpu_sc as plsc

H = 64
W = 72
CHUNK = 128
NUM_LAYERS = 2
N = {'network': 500, 'host': 25000, 'service': 12500, 'data': 12000}
RELS = [('hn', 'host', 'network', 50000), ('nh', 'network', 'host', 50000),
        ('hs', 'host', 'service', 200000), ('hd', 'host', 'data', 200000),
        ('sh', 'service', 'host', 150000), ('dh', 'data', 'host', 150000)]

_N_TILES = 32


def _ceil_to(x, m):
    return (x + m - 1) // m * m


@functools.lru_cache(maxsize=None)
def _edge_kernel(e_pad, n_acc, chunk, nbuf):
    per_tile = e_pad // _N_TILES
    n_chunks = per_tile // chunk
    rows_pt = n_acc // 16
    mesh = plsc.VectorSubcoreMesh(core_axis_name="c", subcore_axis_name="s")

    def body(hl_hbm, hr_hbm, src_hbm, dst_hbm, att_hbm, out_hbm,
             acc_sh, *scr):
        buf_refs, rest = scr[:4 * nbuf], scr[4 * nbuf:]
        att_v, zero_v = rest[0], rest[1]
        sems = rest[2:]
        c = lax.axis_index("c")
        s = lax.axis_index("s")
        pltpu.sync_copy(att_hbm, att_v)
        zeros16 = jnp.zeros((16,), jnp.float32)

        def zrow(i, carry):
            for off in (0, 16, 32, 48, W - 16):
                zero_v[i, pl.ds(off, 16)] = zeros16
            return carry
        lax.fori_loop(0, 16, zrow, 0)

        row0 = s * rows_pt

        def zacc(j, carry):
            pltpu.sync_copy(zero_v, acc_sh.at[pl.ds(row0 + j * 16, 16)])
            return carry
        lax.fori_loop(0, rows_pt // 16, zacc, 0)
        plsc.subcore_barrier()

        tile_base = (c * 16 + s) * per_tile
        eidx0 = lax.iota(jnp.int32, 16)
        att_vecs = [att_v[pl.ds(k * 16, 16)] for k in range(H // 16)]

        sides = tuple(
            buf_refs[4 * j:4 * j + 4] + sems[2 * j:2 * j + 2]
            for j in range(nbuf))

        def issue(side, ci):
            src_v, dst_v, hl_rows, hr_rows, s0, s1 = side
            base = tile_base + ci * chunk
            pltpu.sync_copy(src_hbm.at[pl.ds(base, chunk)], src_v)
            pltpu.sync_copy(dst_hbm.at[pl.ds(base, chunk)], dst_v)
            pltpu.async_copy(hl_hbm.at[src_v], hl_rows, s0)
            pltpu.async_copy(hr_hbm.at[dst_v], hr_rows, s1)

        def wait_side(side):
            src_v, dst_v, hl_rows, hr_rows, s0, s1 = side
            pltpu.make_async_copy(hl_hbm.at[src_v], hl_rows, s0).wait()
            pltpu.make_async_copy(hr_hbm.at[dst_v], hr_rows, s1).wait()

        def compute_scatter(side):
            src_v, dst_v, hl_rows, hr_rows, s0, s1 = side

            def group(g, carry2):
                e_idx = eidx0 + g * 16
                exs = jnp.zeros((16,), jnp.float32)
                for e in range(16):
                    e_abs = g * 16 + e
                    avs = [hl_rows[e_abs, pl.ds(k * 16, 16)]
                           for k in range(H // 16)]
                    ps = []
                    for k in range(H // 16):
                        sm = avs[k] + hr_rows[e_abs, pl.ds(k * 16, 16)]
                        z = jnp.maximum(sm, 0.2 * sm)
                        ps.append(z * att_vecs[k])
                    t = (ps[0] + ps[1]) + (ps[2] + ps[3])
                    logit = jnp.sum(t)
                    ex = jnp.exp(jnp.broadcast_to(logit, (16,)))
                    for k in range(H // 16):
                        hl_rows[e_abs, pl.ds(k * 16, 16)] = avs[k] * ex
                    exs = jnp.where(eidx0 == e, ex, exs)
                f_idx = jnp.full((16,), H, jnp.int32)
                plsc.store_scatter(hl_rows, [e_idx, f_idx], exs)
                return carry2
            lax.fori_loop(0, chunk // 16, group, 0)
            pltpu.sync_copy(hl_rows, acc_sh.at[dst_v], add=True)

        if nbuf == 2:
            n_half = n_chunks // 2
            issue(sides[0], 0)

            def pair_body(i2, carry):
                wait_side(sides[0])
                issue(sides[1], 2 * i2 + 1)
                compute_scatter(sides[0])
                wait_side(sides[1])

                @pl.when(i2 < n_half - 1)
                def _():
                    issue(sides[0], 2 * i2 + 2)
                compute_scatter(sides[1])
                return carry
            lax.fori_loop(0, n_half, pair_body, 0)
        else:
            def chunk_body(i, carry):
                issue(sides[0], i)
                wait_side(sides[0])
                compute_scatter(sides[0])
                return carry
            lax.fori_loop(0, n_chunks, chunk_body, 0)
        plsc.subcore_barrier()

        def drain(j, carry):
            r = row0 + j * 16
            pltpu.sync_copy(acc_sh.at[pl.ds(r, 16)], out_hbm.at[c, pl.ds(r, 16)])
            return carry
        lax.fori_loop(0, rows_pt // 16, drain, 0)

    return pl.kernel(
        body,
        out_type=jax.ShapeDtypeStruct((2, n_acc, W), jnp.float32),
        mesh=mesh,
        compiler_params=pltpu.CompilerParams(
            needs_layout_passes=False, use_tc_tiling_on_sc=False),
        scratch_types=(
            [pltpu.VMEM_SHARED((n_acc, W), jnp.float32)]
            + [pltpu.VMEM((chunk,), jnp.int32),
               pltpu.VMEM((chunk,), jnp.int32),
               pltpu.VMEM((chunk, W), jnp.float32),
               pltpu.VMEM((chunk, H), jnp.float32)] * nbuf
            + [pltpu.VMEM((H,), jnp.float32),
               pltpu.VMEM((16, W), jnp.float32)]
            + [pltpu.SemaphoreType.DMA] * (2 * nbuf)
        ),
    )


def _gatv2_sc(h_src, h_dst, src, dst, Wl, Wr, att, b, num_dst):
    n_src = h_src.shape[0]
    e = src.shape[0]
    n_acc = _ceil_to(num_dst + 1, 256)
    big_acc = n_acc * W > 1500 * 1024
    chunk = 112 if big_acc else CHUNK
    nbuf = 1
    e_pad = _ceil_to(e, nbuf * _N_TILES * chunk)

    hl = h_src @ Wl
    hl_ext = jnp.concatenate(
        [hl, jnp.ones((n_src, 1), jnp.float32),
         jnp.zeros((n_src, W - H - 1), jnp.float32)], axis=1)
    hr = h_dst @ Wr
    src_p = jnp.concatenate([src, jnp.zeros((e_pad - e,), jnp.int32)])
    dst_p = jnp.concatenate(
        [dst, jnp.full((e_pad - e,), num_dst, jnp.int32)])

    acc = _edge_kernel(e_pad, n_acc, chunk, nbuf)(hl_ext, hr, src_p, dst_p, att)
    tot = acc[0] + acc[1]
    den = tot[:num_dst, H]
    out = tot[:num_dst, :H] / (den + 1e-16)[:, None]
    return out + b


def kernel(x_network, x_host, x_service, x_data, e_hn_src, e_hn_dst, e_nh_src, e_nh_dst, e_hs_src, e_hs_dst, e_hd_src, e_hd_dst, e_sh_src, e_sh_dst, e_dh_src, e_dh_dst, enc_W_network, enc_b_network, enc_W_host, enc_b_host, enc_W_service, enc_b_service, enc_W_data, enc_b_data, conv0_hn_Wl, conv0_hn_Wr, conv0_hn_att, conv0_hn_b, conv0_nh_Wl, conv0_nh_Wr, conv0_nh_att, conv0_nh_b, conv0_hs_Wl, conv0_hs_Wr, conv0_hs_att, conv0_hs_b, conv0_hd_Wl, conv0_hd_Wr, conv0_hd_att, conv0_hd_b, conv0_sh_Wl, conv0_sh_Wr, conv0_sh_att, conv0_sh_b, conv0_dh_Wl, conv0_dh_Wr, conv0_dh_att, conv0_dh_b, conv1_hn_Wl, conv1_hn_Wr, conv1_hn_att, conv1_hn_b, conv1_nh_Wl, conv1_nh_Wr, conv1_nh_att, conv1_nh_b, conv1_hs_Wl, conv1_hs_Wr, conv1_hs_att, conv1_hs_b, conv1_hd_Wl, conv1_hd_Wr, conv1_hd_att, conv1_hd_b, conv1_sh_Wl, conv1_sh_Wr, conv1_sh_att, conv1_sh_b, conv1_dh_Wl, conv1_dh_Wr, conv1_dh_att, conv1_dh_b, val_W1, val_b1, val_W2, val_b2, state_summary):
    fl = dict(locals())
    it = {}
    for rel, s, d, e in RELS:
        it['e_' + rel + '_src'] = fl.pop('e_' + rel + '_src')
        it['e_' + rel + '_dst'] = fl.pop('e_' + rel + '_dst')

    h = {}
    for nt in N:
        h[nt] = jax.nn.relu(fl['x_' + nt] @ fl['enc_W_' + nt] + fl['enc_b_' + nt])
    for l in range(NUM_LAYERS):
        new = {nt: jnp.zeros_like(h[nt]) for nt in N}
        for rel, s, d, e in RELS:
            p = 'conv%d_%s_' % (l, rel)
            out = _gatv2_sc(h[s], h[d], it['e_' + rel + '_src'],
                            it['e_' + rel + '_dst'],
                            fl[p + 'Wl'], fl[p + 'Wr'], fl[p + 'att'],
                            fl[p + 'b'], N[d])
            new[d] = new[d] + out
        h = {nt: jax.nn.relu(new[nt]) for nt in N}
    g = jnp.stack([h[nt].mean(axis=0) for nt in N]).mean(axis=0)
    z = jnp.concatenate([g, state_summary])
    v = jax.nn.relu(z @ val_W1 + val_b1) @ val_W2 + val_b2
    return v.squeeze(-1)
